# baseline (speedup 1.0000x reference)
import functools

import jax
import jax.numpy as jnp
from jax import lax
from jax.experimental import pallas as pl
from jax.experimental.pallas import tpu as pltpu
from jax.experimental.pallas import tpu_sc as plsc

_DPAD = 384
_NT = _DPAD // 128
_LPAD = 56
_GRP = 4


@functools.lru_cache(maxsize=None)
def _make_gather(b: int):
    info = plsc.get_sparse_core_info()
    nc = info.num_cores
    nw = nc * info.num_subcores
    per_w = b // nw
    plane = b * _LPAD

    mesh = plsc.VectorSubcoreMesh(core_axis_name="c", subcore_axis_name="s")

    @functools.partial(
        pl.kernel,
        mesh=mesh,
        out_type=jax.ShapeDtypeStruct((_NT * plane, 128), jnp.float32),
        scratch_types=[
            pltpu.VMEM((per_w, _LPAD), jnp.int32),
            pltpu.VMEM((2, _NT, _LPAD), jnp.int32),
            pltpu.VMEM((2, _NT, _LPAD, 128), jnp.float32),
            pltpu.SemaphoreType.DMA,
            pltpu.SemaphoreType.DMA,
            pltpu.SemaphoreType.DMA,
            pltpu.SemaphoreType.DMA,
        ],
    )
    def gather_kernel(table_hbm, idx_hbm, out_hbm, idx_v, jb, rows,
                      sg0, sg1, so0, so1):
        wid = lax.axis_index("s") * nc + lax.axis_index("c")
        base = wid * per_w
        sg = (sg0, sg1)
        so = (so0, so1)

        pltpu.sync_copy(idx_hbm.at[pl.ds(base, per_w)], idx_v)

        def build(j, p):
            for off in (0, 16, 32, _LPAD - 16):
                ids = idx_v[j, pl.ds(off, 16)]
                v3 = ids * _NT
                for t in range(_NT):
                    jb[p, t, pl.ds(off, 16)] = v3 + t

        def fire(p):
            return [
                pltpu.async_copy(table_hbm.at[jb.at[p, t]],
                                 rows.at[p, t], sg[p])
                for t in range(_NT)
            ]

        gathers = [None, None]
        outs = [None, None]
        build(0, 0)
        gathers[0] = fire(0)
        for j in range(per_w):
            p = j % 2
            q = (j + 1) % 2
            if j + 1 < per_w:
                build(j + 1, q)
            for h in gathers[p]:
                h.wait()
            if j + 1 < per_w:
                if outs[q] is not None:
                    for h in outs[q]:
                        h.wait()
                gathers[q] = fire(q)
            row0 = (base + j) * _LPAD
            outs[p] = [
                pltpu.async_copy(
                    rows.at[p, t],
                    out_hbm.at[pl.ds(t * plane + row0, _LPAD)],
                    so[p])
                for t in range(_NT)
            ]
        for hs in outs:
            if hs is not None:
                for h in hs:
                    h.wait()

    return gather_kernel


def _conv_body(p0, p1, p2, out_ref, l, dim):
    planes = (p0, p1, p2)
    for t in range(_NT):
        w = min(dim - 128 * t, 128)
        if w <= 0:
            break
        x = planes[t][...].reshape(_GRP * _LPAD, 128)
        for g in range(_GRP):
            out_ref[g, :, pl.ds(128 * t, w)] = x[_LPAD * g:_LPAD * g + l, :w]


@functools.lru_cache(maxsize=None)
def _make_convert(b: int, l: int, dim: int):
    blk = _GRP * _LPAD * 128
    blocks_per_plane = b // _GRP

    def plane_spec(t):
        return pl.BlockSpec((blk,),
                            lambda i, t=t: (t * blocks_per_plane + i,))

    return pl.pallas_call(
        functools.partial(_conv_body, l=l, dim=dim),
        grid=(b // _GRP,),
        in_specs=[plane_spec(t) for t in range(_NT)],
        out_specs=pl.BlockSpec((_GRP, l, dim), lambda i: (i, 0, 0)),
        out_shape=jax.ShapeDtypeStruct((b, l, dim), jnp.float32),
    )


def kernel(table, word_ids):
    b, l = word_ids.shape
    vocab, dim = table.shape
    idx = jnp.pad(word_ids.astype(jnp.int32), ((0, 0), (0, _LPAD - l)))
    t3 = jnp.pad(table, ((0, 0), (0, _DPAD - dim))).reshape(_NT * vocab, 128)
    staged = _make_gather(b)(t3, idx).reshape(-1)
    return _make_convert(b, l, dim)(staged, staged, staged)


# speedup vs baseline: 2.4775x; 2.4775x over previous
import functools

import jax
import jax.numpy as jnp
from jax import lax
from jax.experimental import pallas as pl
from jax.experimental.pallas import tpu as pltpu
from jax.experimental.pallas import tpu_sc as plsc

_DPAD = 384
_NT = _DPAD // 128
_CHUNK = 80


@functools.lru_cache(maxsize=None)
def _make_gather(n_total: int):
    info = plsc.get_sparse_core_info()
    nc = info.num_cores
    nw = nc * info.num_subcores
    per_w = n_total // nw
    n_chunks = per_w // _CHUNK

    mesh = plsc.VectorSubcoreMesh(core_axis_name="c", subcore_axis_name="s")

    @functools.partial(
        pl.kernel,
        mesh=mesh,
        out_type=jax.ShapeDtypeStruct((_NT * n_total, 128), jnp.float32),
        scratch_types=[
            pltpu.VMEM((per_w,), jnp.int32),
            pltpu.VMEM((2, _NT, _CHUNK), jnp.int32),
            pltpu.VMEM((2, _NT, _CHUNK, 128), jnp.float32),
            pltpu.SemaphoreType.DMA,
            pltpu.SemaphoreType.DMA,
            pltpu.SemaphoreType.DMA,
            pltpu.SemaphoreType.DMA,
        ],
    )
    def gather_kernel(table_hbm, idx_hbm, out_hbm, idx_v, jb, rows,
                      sg0, sg1, so0, so1):
        wid = lax.axis_index("s") * nc + lax.axis_index("c")
        base = wid * per_w
        sg = (sg0, sg1)
        so = (so0, so1)

        pltpu.sync_copy(idx_hbm.at[pl.ds(base, per_w)], idx_v)

        def build(c, p):
            for k in range(_CHUNK // 16):
                ids = idx_v[pl.ds(c * _CHUNK + 16 * k, 16)]
                v3 = ids * _NT
                for t in range(_NT):
                    jb[p, t, pl.ds(16 * k, 16)] = v3 + t

        def fire(p):
            return [
                pltpu.async_copy(table_hbm.at[jb.at[p, t]],
                                 rows.at[p, t], sg[p])
                for t in range(_NT)
            ]

        gathers = [None, None]
        outs = [None, None]
        build(0, 0)
        gathers[0] = fire(0)
        for c in range(n_chunks):
            p = c % 2
            q = (c + 1) % 2
            if c + 1 < n_chunks:
                build(c + 1, q)
            for h in gathers[p]:
                h.wait()
            if c + 1 < n_chunks:
                if outs[q] is not None:
                    for h in outs[q]:
                        h.wait()
                gathers[q] = fire(q)
            outs[p] = [
                pltpu.async_copy(
                    rows.at[p, t],
                    out_hbm.at[pl.ds(t * n_total + base + c * _CHUNK, _CHUNK)],
                    so[p])
                for t in range(_NT)
            ]
        for hs in outs:
            if hs is not None:
                for h in hs:
                    h.wait()

    return gather_kernel


def kernel(table, word_ids):
    b, l = word_ids.shape
    vocab, dim = table.shape
    n = b * l
    idx = word_ids.reshape(-1).astype(jnp.int32)
    t3 = jnp.pad(table, ((0, 0), (0, _DPAD - dim))).reshape(_NT * vocab, 128)
    staged = _make_gather(n)(t3, idx)
    out = staged.reshape(_NT, b, l, 128).transpose(1, 2, 0, 3)
    return out.reshape(b, l, _DPAD)[:, :, :dim]


# speedup vs baseline: 2.8202x; 1.1383x over previous
import functools

import jax
import jax.numpy as jnp
from jax import lax
from jax.experimental import pallas as pl
from jax.experimental.pallas import tpu as pltpu
from jax.experimental.pallas import tpu_sc as plsc

_DPAD = 384
_NT = _DPAD // 128
_CHUNK = 32


@functools.lru_cache(maxsize=None)
def _make_gather(n_total: int):
    info = plsc.get_sparse_core_info()
    nc = info.num_cores
    nw = nc * info.num_subcores
    per_w = n_total // nw
    n_chunks = per_w // _CHUNK

    mesh = plsc.VectorSubcoreMesh(core_axis_name="c", subcore_axis_name="s")

    @functools.partial(
        pl.kernel,
        mesh=mesh,
        compiler_params=pltpu.CompilerParams(needs_layout_passes=False),
        out_type=jax.ShapeDtypeStruct((_NT * n_total, 128), jnp.float32),
        scratch_types=[
            pltpu.VMEM((per_w,), jnp.int32),
            pltpu.VMEM((2, _NT * _CHUNK), jnp.int32),
            pltpu.VMEM((2, _NT * _CHUNK, 128), jnp.float32),
            pltpu.SemaphoreType.DMA,
            pltpu.SemaphoreType.DMA,
            pltpu.SemaphoreType.DMA,
            pltpu.SemaphoreType.DMA,
        ],
    )
    def gather_kernel(table_hbm, idx_hbm, out_hbm, idx_v, jb, rows,
                      sg0, sg1, so0, so1):
        wid = lax.axis_index("s") * nc + lax.axis_index("c")
        base = wid * per_w
        sg = (sg0, sg1)
        so = (so0, so1)

        pltpu.sync_copy(idx_hbm.at[pl.ds(base, per_w)], idx_v)

        iota = lax.iota(jnp.int32, 16)
        i3 = iota * _NT

        def build(c, p):
            for k in range(_CHUNK // 16):
                ids = idx_v[pl.ds(c * _CHUNK + 16 * k, 16)]
                v3 = ids * _NT
                for t in range(_NT):
                    plsc.store_scatter(jb.at[p], [i3 + (_NT * 16 * k + t)],
                                       v3 + t)

        def fire(p):
            return pltpu.async_copy(table_hbm.at[jb.at[p]], rows.at[p], sg[p])

        gathers = [None, None]
        outs = [None, None]
        build(0, 0)
        gathers[0] = fire(0)
        for c in range(n_chunks):
            p = c % 2
            q = (c + 1) % 2
            if c + 1 < n_chunks:
                build(c + 1, q)
            gathers[p].wait()
            if c + 1 < n_chunks:
                if outs[q] is not None:
                    outs[q].wait()
                gathers[q] = fire(q)
            outs[p] = pltpu.async_copy(
                rows.at[p],
                out_hbm.at[pl.ds(_NT * (base + c * _CHUNK), _NT * _CHUNK)],
                so[p])
        for o in outs:
            if o is not None:
                o.wait()

    return gather_kernel


def kernel(table, word_ids):
    b, l = word_ids.shape
    vocab, dim = table.shape
    n = b * l
    idx = word_ids.reshape(-1).astype(jnp.int32)
    t3 = jnp.pad(table, ((0, 0), (0, _DPAD - dim))).reshape(_NT * vocab, 128)
    staged = _make_gather(n)(t3, idx)
    return staged.reshape(b, l, _DPAD)[:, :, :dim]


# speedup vs baseline: 2.8884x; 1.0242x over previous
import functools

import jax
import jax.numpy as jnp
from jax import lax
from jax.experimental import pallas as pl
from jax.experimental.pallas import tpu as pltpu
from jax.experimental.pallas import tpu_sc as plsc

_DPAD = 384
_NT = _DPAD // 128
_CHUNK = 40


@functools.lru_cache(maxsize=None)
def _make_gather(n_total: int):
    info = plsc.get_sparse_core_info()
    nc = info.num_cores
    nw = nc * info.num_subcores
    per_w = n_total // nw
    n_chunks = per_w // _CHUNK

    mesh = plsc.VectorSubcoreMesh(core_axis_name="c", subcore_axis_name="s")

    @functools.partial(
        pl.kernel,
        mesh=mesh,
        compiler_params=pltpu.CompilerParams(needs_layout_passes=False),
        out_type=jax.ShapeDtypeStruct((_NT * n_total, 128), jnp.float32),
        scratch_types=[
            pltpu.VMEM((per_w,), jnp.int32),
            pltpu.VMEM((2, _NT * _CHUNK), jnp.int32),
            pltpu.VMEM((2, _NT * _CHUNK, 128), jnp.float32),
            pltpu.SemaphoreType.DMA,
            pltpu.SemaphoreType.DMA,
            pltpu.SemaphoreType.DMA,
            pltpu.SemaphoreType.DMA,
        ],
    )
    def gather_kernel(table_hbm, idx_hbm, out_hbm, idx_v, jb, rows,
                      sg0, sg1, so0, so1):
        wid = lax.axis_index("s") * nc + lax.axis_index("c")
        base = wid * per_w
        sg = (sg0, sg1)
        so = (so0, so1)

        pltpu.sync_copy(idx_hbm.at[pl.ds(base, per_w)], idx_v)

        iota = lax.iota(jnp.int32, 16)
        i3 = iota * _NT

        def build(c, p):
            for off in (0, 16, _CHUNK - 16):
                ids = idx_v[pl.ds(c * _CHUNK + off, 16)]
                v3 = ids * _NT
                for t in range(_NT):
                    plsc.store_scatter(jb.at[p], [i3 + (_NT * off + t)],
                                       v3 + t)

        def fire(p):
            return pltpu.async_copy(table_hbm.at[jb.at[p]], rows.at[p], sg[p])

        gathers = [None, None]
        outs = [None, None]
        build(0, 0)
        gathers[0] = fire(0)
        for c in range(n_chunks):
            p = c % 2
            q = (c + 1) % 2
            if c + 1 < n_chunks:
                build(c + 1, q)
            gathers[p].wait()
            if c + 1 < n_chunks:
                if outs[q] is not None:
                    outs[q].wait()
                gathers[q] = fire(q)
            outs[p] = pltpu.async_copy(
                rows.at[p],
                out_hbm.at[pl.ds(_NT * (base + c * _CHUNK), _NT * _CHUNK)],
                so[p])
        for o in outs:
            if o is not None:
                o.wait()

    return gather_kernel


def kernel(table, word_ids):
    b, l = word_ids.shape
    vocab, dim = table.shape
    n = b * l
    idx = word_ids.reshape(-1).astype(jnp.int32)
    t3 = jnp.pad(table, ((0, 0), (0, _DPAD - dim))).reshape(_NT * vocab, 128)
    staged = _make_gather(n)(t3, idx)
    return staged.reshape(b, l, _DPAD)[:, :, :dim]
